# baseline (speedup 1.0000x reference)
import functools

import jax
import jax.numpy as jnp
from jax import lax
from jax.experimental import pallas as pl
from jax.experimental.pallas import tpu as pltpu
from jax.experimental.pallas import tpu_sc as plsc

NC, NS = 2, 16
NW = NC * NS
CHUNK_ROWS = 8
NBUF = 4
LANES = 16
UNROLL = 8


def kernel(x, pos_table):
    batch, seq, d = x.shape
    n_rows = batch * seq
    rows_w = n_rows // NW
    nchunk = rows_w // CHUNK_ROWS
    cn = CHUNK_ROWS * d
    w_per_b = seq // rows_w
    xf = x.reshape(-1)
    pf = pos_table[:seq].reshape(-1)
    mesh = plsc.VectorSubcoreMesh(core_axis_name="c", subcore_axis_name="s")

    @functools.partial(
        pl.kernel,
        mesh=mesh,
        out_type=jax.ShapeDtypeStruct((n_rows * d,), jnp.float32),
        scratch_types=[
            pltpu.VMEM((NBUF, cn), jnp.float32),
            pltpu.VMEM((NBUF, cn), jnp.float32),
            pltpu.SemaphoreType.DMA((NBUF,)),
            pltpu.SemaphoreType.DMA((NBUF,)),
            pltpu.SemaphoreType.DMA((NBUF,)),
        ],
    )
    def sc_k(x_hbm, p_hbm, o_hbm, xbuf, pbuf, six, sip, sout):
        wid = lax.axis_index("s") * NC + lax.axis_index("c")
        x0 = wid * (rows_w * d)
        p0 = (wid % w_per_b) * (rows_w * d)

        def issue_in(g, b):
            pltpu.async_copy(x_hbm.at[pl.ds(x0 + g * cn, cn)], xbuf.at[b], six.at[b])
            pltpu.async_copy(p_hbm.at[pl.ds(p0 + g * cn, cn)], pbuf.at[b], sip.at[b])

        def wait_in(b):
            pltpu.make_async_copy(
                x_hbm.at[pl.ds(x0, cn)], xbuf.at[b], six.at[b]
            ).wait()
            pltpu.make_async_copy(
                p_hbm.at[pl.ds(p0, cn)], pbuf.at[b], sip.at[b]
            ).wait()

        def wait_out(b):
            pltpu.make_async_copy(
                xbuf.at[b], o_hbm.at[pl.ds(x0, cn)], sout.at[b]
            ).wait()

        issue_in(0, 0)
        issue_in(1, 1)

        @pl.loop(0, nchunk, step=NBUF)
        def _chunks(g):
            for b in range(NBUF):
                gi = g + b
                jb = (b + 2) % NBUF
                jj = gi + 2

                @pl.when(jj < nchunk)
                def _():
                    @pl.when(jj >= NBUF)
                    def _():
                        wait_out(jb)

                    issue_in(jj, jb)

                wait_in(b)

                @pl.loop(0, cn, step=LANES * UNROLL)
                def _vec(c):
                    for j in range(UNROLL):
                        sl = pl.ds(c + j * LANES, LANES)
                        plsc.addupdate(xbuf.at[b, sl], pbuf[b, sl])

                pltpu.async_copy(
                    xbuf.at[b], o_hbm.at[pl.ds(x0 + gi * cn, cn)], sout.at[b]
                )

        for b in range(NBUF):
            wait_out(b)

    return sc_k(xf, pf).reshape(batch, seq, d)


# speedup vs baseline: 5.4443x; 5.4443x over previous
import jax
import jax.numpy as jnp
from jax.experimental import pallas as pl
from jax.experimental.pallas import tpu as pltpu


def _add_pos_kernel(x_ref, pos_ref, out_ref):
    out_ref[...] = x_ref[...] + pos_ref[...][None, :, :]


def kernel(x, pos_table):
    batch, seq, d_model = x.shape
    bs = 1024
    bh = batch // 2
    grid = (seq // bs, 2)
    return pl.pallas_call(
        _add_pos_kernel,
        grid=grid,
        in_specs=[
            pl.BlockSpec((bh, bs, d_model), lambda i, j: (j, i, 0)),
            pl.BlockSpec((bs, d_model), lambda i, j: (i, 0)),
        ],
        out_specs=pl.BlockSpec((bh, bs, d_model), lambda i, j: (j, i, 0)),
        out_shape=jax.ShapeDtypeStruct((batch, seq, d_model), x.dtype),
        compiler_params=pltpu.CompilerParams(
            dimension_semantics=("parallel", "parallel"),
        ),
    )(x, pos_table[:seq])
